# Initial kernel scaffold; baseline (speedup 1.0000x reference)
import jax
import jax.numpy as jnp
from jax.experimental import pallas as pl


def kernel(x, edge_index, edge_attr, batch, frag_x, frag_edge_index, frag_edge_attr, frag_batch, motif_x, junction_edge_index, junction_edge_attr, junction_batch, params):
    raise NotImplementedError("write your pallas kernel here")


# speedup vs baseline: 3.7973x; 3.7973x over previous
import functools

import jax
import jax.numpy as jnp
from jax import lax
from jax.experimental import pallas as pl
from jax.experimental.pallas import tpu as pltpu
from jax.experimental.pallas import tpu_sc as plsc

_H = 128
_TM = 512
_NEG = -1e30


def _act(x, kind):
    if kind == 'none':
        return x
    if kind == 'leaky':
        return jnp.where(x >= 0, x, 0.01 * x)
    if kind == 'leaky3':
        return jnp.where(x >= 0, x, 0.001 * x)
    if kind == 'tanh':
        return jnp.tanh(x)
    if kind == 'relu':
        return jnp.maximum(x, 0.0)
    if kind == 'elu':
        return jnp.where(x > 0, x, jnp.exp(x) - 1.0)
    raise ValueError(kind)


def _pad_rows(a, m):
    return a if a.shape[0] == m else jnp.pad(a, ((0, m - a.shape[0]),) + ((0, 0),) * (a.ndim - 1))


def _pad1(a, m, val=0):
    return a if a.shape[0] == m else jnp.pad(a, (0, m - a.shape[0]), constant_values=val)


def _rup(n, m):
    return ((n + m - 1) // m) * m


def _linN_body(*refs, nx, act):
    xs = refs[:nx]
    ws = refs[nx:2 * nx]
    b = refs[2 * nx]
    y = refs[2 * nx + 1]
    acc = jnp.dot(xs[0][...], ws[0][...], preferred_element_type=jnp.float32, precision=lax.Precision.HIGHEST)
    for i in range(1, nx):
        acc = acc + jnp.dot(xs[i][...], ws[i][...], preferred_element_type=jnp.float32, precision=lax.Precision.HIGHEST)
    y[...] = _act(acc + b[...], act)


def _linN(xs, ws, b, act):
    m = xs[0].shape[0]
    hd = ws[0].shape[1]
    grid = (m // _TM,)
    in_specs = [pl.BlockSpec((_TM, x.shape[1]), lambda i: (i, 0)) for x in xs]
    in_specs += [pl.BlockSpec(w.shape, lambda i: (0, 0)) for w in ws]
    in_specs.append(pl.BlockSpec(b.shape, lambda i: (0,)))
    return pl.pallas_call(
        functools.partial(_linN_body, nx=len(xs), act=act),
        grid=grid,
        in_specs=in_specs,
        out_specs=pl.BlockSpec((_TM, hd), lambda i: (i, 0)),
        out_shape=jax.ShapeDtypeStruct((m, hd), jnp.float32),
    )(*xs, *ws, b)


def _hd_body(*refs, shared1, has2, shared2, act0, has_wc, act1):
    i = 0
    x1 = refs[i]; i += 1
    x2 = None
    if has2:
        x2 = refs[i]; i += 1
    w1 = refs[i]; i += 1
    w2 = None
    if has2:
        w2 = refs[i]; i += 1
    b = refs[i]; i += 1
    wc = None
    if has_wc:
        wc = refs[i]; i += 1
    a = refs[i]; i += 1
    y = refs[i]; p = refs[i + 1]
    x1v = x1[...] if shared1 else x1[0]
    acc = jnp.dot(x1v, w1[0], preferred_element_type=jnp.float32, precision=lax.Precision.HIGHEST)
    if has2:
        x2v = x2[...] if shared2 else x2[0]
        acc = acc + jnp.dot(x2v, w2[0], preferred_element_type=jnp.float32, precision=lax.Precision.HIGHEST)
    y0 = _act(acc + b[0, 0], act0)
    if has_wc:
        y0 = _act(jnp.dot(y0, wc[0], preferred_element_type=jnp.float32, precision=lax.Precision.HIGHEST), act1)
    y[...] = y0[None]
    p[...] = jnp.dot(y0, a[0], preferred_element_type=jnp.float32, precision=lax.Precision.HIGHEST)[None]


def _hd(x1, x2, w1, w2, b, wc, act0, act1, a):
    shared1 = x1.ndim == 2
    has2 = x2 is not None
    shared2 = has2 and x2.ndim == 2
    has_wc = wc is not None
    m = x1.shape[0] if shared1 else x1.shape[1]
    hout = wc.shape[2] if has_wc else w1.shape[2]
    grid = (2, m // _TM)

    in_specs = []
    args = []
    if shared1:
        k1 = x1.shape[1]
        in_specs.append(pl.BlockSpec((_TM, k1), lambda h, i: (i, 0)))
    else:
        k1 = x1.shape[2]
        in_specs.append(pl.BlockSpec((1, _TM, k1), lambda h, i: (h, i, 0)))
    args.append(x1)
    if has2:
        if shared2:
            k2 = x2.shape[1]
            in_specs.append(pl.BlockSpec((_TM, k2), lambda h, i: (i, 0)))
        else:
            k2 = x2.shape[2]
            in_specs.append(pl.BlockSpec((1, _TM, k2), lambda h, i: (h, i, 0)))
        args.append(x2)
    in_specs.append(pl.BlockSpec((1,) + w1.shape[1:], lambda h, i: (h, 0, 0)))
    args.append(w1)
    if has2:
        in_specs.append(pl.BlockSpec((1,) + w2.shape[1:], lambda h, i: (h, 0, 0)))
        args.append(w2)
    in_specs.append(pl.BlockSpec((1, 1, b.shape[1]), lambda h, i: (h, 0, 0)))
    args.append(b.reshape(2, 1, b.shape[1]))
    if has_wc:
        in_specs.append(pl.BlockSpec((1,) + wc.shape[1:], lambda h, i: (h, 0, 0)))
        args.append(wc)
    in_specs.append(pl.BlockSpec((1,) + a.shape[1:], lambda h, i: (h, 0, 0)))
    args.append(a)

    out_specs = [
        pl.BlockSpec((1, _TM, hout), lambda h, i: (h, i, 0)),
        pl.BlockSpec((1, _TM, 8), lambda h, i: (h, i, 0)),
    ]
    out_shape = [
        jax.ShapeDtypeStruct((2, m, hout), jnp.float32),
        jax.ShapeDtypeStruct((2, m, 8), jnp.float32),
    ]
    return pl.pallas_call(
        functools.partial(_hd_body, shared1=shared1, has2=has2, shared2=shared2,
                          act0=act0, has_wc=has_wc, act1=act1),
        grid=grid,
        in_specs=in_specs,
        out_specs=out_specs,
        out_shape=out_shape,
    )(*args)


def _seg_body(table_hbm, srcg_hbm, dst_hbm, pa_hbm, pb_hbm, pe_hbm, e_hbm,
              acc_out, alpha_out,
              pa_v, pb_v, ssum_v, w_v, srcg_c, dst_c, pe_c, alpha_c,
              rows_v, e_v, zbuf, acc_sh, ssum_sh, sem,
              *, ew, ch, nseg_pad, nsrc_off, softmax, has_e, emit_alpha):
    cid = lax.axis_index("c")
    sid = lax.axis_index("s")

    def zrow(j, c):
        def zcol(k, cc):
            zbuf[j, pl.ds(k * 16, 16)] = jnp.zeros((16,), jnp.float32)
            return cc
        return lax.fori_loop(0, 8, zcol, c)
    lax.fori_loop(0, 16, zrow, 0)

    rows_pt = nseg_pad // 16

    def zacc(j, c):
        pltpu.sync_copy(zbuf, acc_sh.at[pl.ds(sid * rows_pt + j * 16, 16)])
        return c
    lax.fori_loop(0, rows_pt // 16, zacc, 0)

    if softmax:
        def zs(j, c):
            pltpu.sync_copy(zbuf.at[0, pl.ds(0, 16)],
                            ssum_sh.at[pl.ds(sid * rows_pt + j * 16, 16)])
            return c
        lax.fori_loop(0, rows_pt // 16, zs, 0)
        pltpu.sync_copy(pa_hbm.at[cid, pl.ds(0, nseg_pad)], pa_v)
        pltpu.sync_copy(pb_hbm.at[cid, pl.ds(0, pb_hbm.shape[1])], pb_v)
    plsc.subcore_barrier()

    nch = ew // ch
    ebase0 = sid * ew
    off = cid * nsrc_off

    if softmax:
        def pass_a(c, carry):
            eb = ebase0 + c * ch
            pltpu.sync_copy(dst_hbm.at[pl.ds(eb, ch)], dst_c)
            pltpu.sync_copy(pe_hbm.at[cid, pl.ds(eb, ch)], pe_c)
            pltpu.sync_copy(srcg_hbm.at[cid, pl.ds(eb, ch)], srcg_c)

            def inner(j, cc):
                o = j * 16
                dv = dst_c[pl.ds(o, 16)]
                sv = srcg_c[pl.ds(o, 16)] - off
                l = (plsc.load_gather(pa_v, [dv])
                     + plsc.load_gather(pb_v, [sv]) + pe_c[pl.ds(o, 16)])
                l = jnp.where(l >= 0, l, l * 0.01)
                w_v[pl.ds(c * ch + o, 16)] = jnp.exp(l)
                return cc
            lax.fori_loop(0, ch // 16, inner, 0)
            pltpu.sync_copy(w_v.at[pl.ds(c * ch, ch)], ssum_sh.at[dst_c], add=True)
            return carry
        lax.fori_loop(0, nch, pass_a, 0)
        plsc.subcore_barrier()
        pltpu.sync_copy(ssum_sh, ssum_v)

    def pass_b(c, carry):
        eb = ebase0 + c * ch
        pltpu.sync_copy(dst_hbm.at[pl.ds(eb, ch)], dst_c)
        pltpu.sync_copy(srcg_hbm.at[cid, pl.ds(eb, ch)], srcg_c)

        if softmax:
            def mka(j, cc):
                o = j * 16
                w16 = w_v[pl.ds(c * ch + o, 16)]
                sv = plsc.load_gather(ssum_v, [dst_c[pl.ds(o, 16)]])
                alpha_c[pl.ds(o, 16)] = w16 / (sv + 1e-9)
                return cc
            lax.fori_loop(0, ch // 16, mka, 0)
        else:
            pltpu.sync_copy(pe_hbm.at[cid, pl.ds(eb, ch)], alpha_c)
        if emit_alpha:
            pltpu.sync_copy(alpha_c, alpha_out.at[cid, pl.ds(eb, ch)])

        pltpu.async_copy(table_hbm.at[srcg_c], rows_v, sem).wait()
        if has_e:
            pltpu.sync_copy(e_hbm.at[cid, pl.ds(eb, ch)], e_v)

        def srow(j, cc):
            av = plsc.load_gather(alpha_c, [jnp.full((16,), j, jnp.int32)])

            def scol(k, c2):
                sl = pl.ds(k * 16, 16)
                if has_e:
                    rows_v[j, sl] = (rows_v[j, sl] + e_v[j, sl]) * av
                else:
                    rows_v[j, sl] = rows_v[j, sl] * av
                return c2
            return lax.fori_loop(0, 8, scol, cc)
        lax.fori_loop(0, ch, srow, 0)
        pltpu.sync_copy(rows_v, acc_sh.at[dst_c], add=True)
        return carry
    lax.fori_loop(0, nch, pass_b, 0)
    plsc.subcore_barrier()

    def dump(j, c):
        r = sid * rows_pt + j * 16
        pltpu.sync_copy(acc_sh.at[pl.ds(r, 16)], acc_out.at[cid, pl.ds(r, 16)])
        return c
    lax.fori_loop(0, rows_pt // 16, dump, 0)


def _seg_call(table2, srcg, dst, pa, pb, pe, e2, *, nseg_pad, nsrc_off, ch,
              softmax, has_e, emit_alpha):
    e_pad = dst.shape[0]
    ew = e_pad // 16
    d = _H
    out_type = [
        jax.ShapeDtypeStruct((2, nseg_pad, d), jnp.float32),
        jax.ShapeDtypeStruct((2, e_pad if emit_alpha else 16), jnp.float32),
    ]
    scratch = [
        pltpu.VMEM((nseg_pad if softmax else 16,), jnp.float32),
        pltpu.VMEM((pb.shape[1] if softmax else 16,), jnp.float32),
        pltpu.VMEM((nseg_pad if softmax else 16,), jnp.float32),
        pltpu.VMEM((ew if softmax else 16,), jnp.float32),
        pltpu.VMEM((ch,), jnp.int32),
        pltpu.VMEM((ch,), jnp.int32),
        pltpu.VMEM((ch if softmax else 16,), jnp.float32),
        pltpu.VMEM((ch,), jnp.float32),
        pltpu.VMEM((ch, d), jnp.float32),
        pltpu.VMEM((ch, d) if has_e else (16, d), jnp.float32),
        pltpu.VMEM((16, d), jnp.float32),
        pltpu.VMEM_SHARED((nseg_pad, d), jnp.float32),
        pltpu.VMEM_SHARED((nseg_pad if softmax else 16,), jnp.float32),
        pltpu.SemaphoreType.DMA,
    ]
    fn = pl.kernel(
        functools.partial(_seg_body, ew=ew, ch=ch, nseg_pad=nseg_pad,
                          nsrc_off=nsrc_off, softmax=softmax, has_e=has_e,
                          emit_alpha=emit_alpha),
        mesh=plsc.VectorSubcoreMesh(core_axis_name="c", subcore_axis_name="s"),
        out_type=out_type,
        scratch_types=scratch,
        compiler_params=pltpu.CompilerParams(needs_layout_passes=False),
    )
    return fn(table2, srcg, dst, pa, pb, pe, e2)


def _att_body(srcg_hbm, dst_hbm, pa_hbm, pb_hbm, pe_hbm,
              alpha_out,
              pa_v, pb_v, ssum_v, w_v, srcg_c, dst_c, pe_c, alpha_c, zbuf,
              ssum_sh,
              *, ew, ch, nseg_pad, nsrc_off):
    cid = lax.axis_index("c")
    sid = lax.axis_index("s")
    zbuf[pl.ds(0, 16)] = jnp.zeros((16,), jnp.float32)
    rows_pt = nseg_pad // 16

    def zs(j, c):
        pltpu.sync_copy(zbuf, ssum_sh.at[pl.ds(sid * rows_pt + j * 16, 16)])
        return c
    lax.fori_loop(0, rows_pt // 16, zs, 0)
    pltpu.sync_copy(pa_hbm.at[cid, pl.ds(0, nseg_pad)], pa_v)
    pltpu.sync_copy(pb_hbm.at[cid, pl.ds(0, pb_hbm.shape[1])], pb_v)
    plsc.subcore_barrier()

    nch = ew // ch
    ebase0 = sid * ew
    off = cid * nsrc_off

    def pass_a(c, carry):
        eb = ebase0 + c * ch
        pltpu.sync_copy(dst_hbm.at[pl.ds(eb, ch)], dst_c)
        pltpu.sync_copy(pe_hbm.at[cid, pl.ds(eb, ch)], pe_c)
        pltpu.sync_copy(srcg_hbm.at[cid, pl.ds(eb, ch)], srcg_c)

        def inner(j, cc):
            o = j * 16
            dv = dst_c[pl.ds(o, 16)]
            sv = srcg_c[pl.ds(o, 16)] - off
            l = (plsc.load_gather(pa_v, [dv])
                 + plsc.load_gather(pb_v, [sv]) + pe_c[pl.ds(o, 16)])
            l = jnp.where(l >= 0, l, l * 0.01)
            w_v[pl.ds(c * ch + o, 16)] = jnp.exp(l)
            return cc
        lax.fori_loop(0, ch // 16, inner, 0)
        pltpu.sync_copy(w_v.at[pl.ds(c * ch, ch)], ssum_sh.at[dst_c], add=True)
        return carry
    lax.fori_loop(0, nch, pass_a, 0)
    plsc.subcore_barrier()
    pltpu.sync_copy(ssum_sh, ssum_v)

    def norm(c, carry):
        eb = ebase0 + c * ch
        pltpu.sync_copy(dst_hbm.at[pl.ds(eb, ch)], dst_c)

        def mka(j, cc):
            o = j * 16
            w16 = w_v[pl.ds(c * ch + o, 16)]
            sv = plsc.load_gather(ssum_v, [dst_c[pl.ds(o, 16)]])
            alpha_c[pl.ds(o, 16)] = w16 / (sv + 1e-9)
            return cc
        lax.fori_loop(0, ch // 16, mka, 0)
        pltpu.sync_copy(alpha_c, alpha_out.at[cid, pl.ds(eb, ch)])
        return carry
    lax.fori_loop(0, nch, norm, 0)


def _att_call(srcg, dst, pa, pb, pe, *, nseg_pad, nsrc_off, ch):
    e_pad = dst.shape[0]
    ew = e_pad // 16
    scratch = [
        pltpu.VMEM((nseg_pad,), jnp.float32),
        pltpu.VMEM((pb.shape[1],), jnp.float32),
        pltpu.VMEM((nseg_pad,), jnp.float32),
        pltpu.VMEM((ew,), jnp.float32),
        pltpu.VMEM((ch,), jnp.int32),
        pltpu.VMEM((ch,), jnp.int32),
        pltpu.VMEM((ch,), jnp.float32),
        pltpu.VMEM((ch,), jnp.float32),
        pltpu.VMEM((16,), jnp.float32),
        pltpu.VMEM_SHARED((nseg_pad,), jnp.float32),
    ]
    fn = pl.kernel(
        functools.partial(_att_body, ew=ew, ch=ch, nseg_pad=nseg_pad,
                          nsrc_off=nsrc_off),
        mesh=plsc.VectorSubcoreMesh(core_axis_name="c", subcore_axis_name="s"),
        out_type=[jax.ShapeDtypeStruct((2, e_pad), jnp.float32)],
        scratch_types=scratch,
        compiler_params=pltpu.CompilerParams(needs_layout_passes=False),
    )
    res = fn(srcg, dst, pa, pb, pe)
    return res[0] if isinstance(res, (tuple, list)) else res


def _stack(heads, *path):
    def get(h):
        v = h
        for p in path:
            v = v[p]
        return v
    return jnp.stack([get(heads[0]), get(heads[1])])


def _a8(cols):
    a = jnp.zeros((2, _H, 8), jnp.float32)
    for i, c in enumerate(cols):
        a = a.at[:, :, i].set(c)
    return a


def _afp_core(heads, h, hp, e2, pe_list, srcg_e, dst_e, srcg_n, dst_n,
              pe_pool, n_pad, g_pad, ch_e, ch_n, emit_alpha):
    la = len(heads[0]['atom'])
    lm = len(heads[0]['mol'])
    dummy_e = jnp.zeros((2, 16, _H), jnp.float32)
    zerob = jnp.zeros((2, _H), jnp.float32)

    dummy16 = jnp.zeros((2, 16), jnp.float32)
    for l in range(la):
        if n_pad >= 8192:
            alpha_e = _att_call(srcg_e, dst_e, hp[:, :, 0], hp[:, :, 1],
                                pe_list[l], nseg_pad=n_pad, nsrc_off=n_pad,
                                ch=256)
            acc, _ = _seg_call(h.reshape(2 * n_pad, _H), srcg_e, dst_e,
                               dummy16, dummy16, alpha_e, e2,
                               nseg_pad=n_pad, nsrc_off=n_pad, ch=128,
                               softmax=False, has_e=True, emit_alpha=False)
        else:
            acc, _ = _seg_call(h.reshape(2 * n_pad, _H), srcg_e, dst_e,
                               hp[:, :, 0], hp[:, :, 1], pe_list[l], e2,
                               nseg_pad=n_pad, nsrc_off=n_pad, ch=ch_e,
                               softmax=True, has_e=True, emit_alpha=False)
        if l + 1 < la:
            nxt = _a8([_stack(heads, 'atom', l + 1, 'a')[:, :_H, 0],
                       _stack(heads, 'atom', l + 1, 'a')[:, _H:, 0]])
        else:
            nxt = _a8([_stack(heads, 'mol', ml, 'a')[:, _H:, 0] for ml in range(lm)])
        h, hp = _hd(h, acc, _stack(heads, 'atom', l, 'W_u'),
                    _stack(heads, 'atom', l, 'W_m'), zerob, None,
                    'elu', 'none', nxt)

    zs_g = jnp.zeros((2, g_pad), jnp.float32)
    zs_n = jnp.zeros((2, n_pad), jnp.float32)
    s, _ = _seg_call(h.reshape(2 * n_pad, _H), srcg_n, dst_n, zs_g, zs_n,
                     pe_pool, dummy_e, nseg_pad=g_pad, nsrc_off=n_pad,
                     ch=ch_n, softmax=True, has_e=False, emit_alpha=False)

    sp, _ = _hd(s, None, _a8([_stack(heads, 'mol', 0, 'a')[:, :_H, 0]]), None,
                jnp.zeros((2, 8), jnp.float32), None, 'none', 'none',
                jnp.zeros((2, 8, 8), jnp.float32))
    alpha = None
    for l in range(lm):
        acc_c, alpha = _seg_call(h.reshape(2 * n_pad, _H), srcg_n, dst_n,
                                 sp[:, :, 0], hp[:, :, l], pe_pool, dummy_e,
                                 nseg_pad=g_pad, nsrc_off=n_pad, ch=ch_n,
                                 softmax=True, has_e=False,
                                 emit_alpha=(emit_alpha and l == lm - 1))
        if l + 1 < lm:
            nxt = _a8([_stack(heads, 'mol', l + 1, 'a')[:, :_H, 0]])
        else:
            nxt = jnp.zeros((2, _H, 8), jnp.float32)
        s, sp = _hd(s, acc_c, _stack(heads, 'mol', l, 'W_s'),
                    _stack(heads, 'mol', l, 'W_c'), zerob, None,
                    'elu', 'none', nxt)
    return s, alpha


def _edge_pe(pep, e_real, la):
    ev = jnp.arange(pep.shape[1], dtype=jnp.int32) < e_real
    return [jnp.where(ev[None, :], pep[:, :, l], _NEG) for l in range(la)]


def kernel(x, edge_index, edge_attr, batch, frag_x, frag_edge_index,
           frag_edge_attr, frag_batch, motif_x, junction_edge_index,
           junction_edge_attr, junction_batch, params):
    n, e, nm = x.shape[0], edge_index.shape[1], 500
    nf, ef = frag_x.shape[0], frag_edge_index.shape[1]
    f, ej = motif_x.shape[0], junction_edge_index.shape[1]

    n_pad = _rup(n, 2048)
    e_pad = _rup(e, 16 * 256)
    nf_pad = _rup(nf, 2048)
    ef_pad = _rup(ef, 2048)
    f_pad = _rup(f, 2048)
    ej_pad = _rup(ej, 2048)
    g_pad = _rup(nm, 512)

    po, pf, pj, pp = params['origin'], params['frag'], params['junction'], params['pred']
    oh, fh = po['heads'], pf['heads']
    jh = [hp['afp'] for hp in pj['heads']]

    def idx2(src, m_pad, off):
        sp_ = _pad1(src, m_pad, 0)
        return jnp.stack([sp_, sp_ + off])

    def pool_pe(m_pad, real):
        v = jnp.arange(m_pad, dtype=jnp.int32) < real
        return jnp.broadcast_to(jnp.where(v, 0.0, _NEG)[None], (2, m_pad)).astype(jnp.float32)

    a8o0 = _a8([_stack(oh, 'atom', 0, 'a')[:, :_H, 0],
                _stack(oh, 'atom', 0, 'a')[:, _H:, 0]])
    h0, hp0 = _hd(_pad_rows(x, n_pad), None,
                  jnp.stack([po['node']['W']] * 2), None,
                  jnp.stack([po['node']['b']] * 2),
                  _stack(oh, 'W_in'), 'leaky', 'tanh', a8o0)
    a8oe = _a8([_stack(oh, 'atom', l, 'a')[:, _H:, 0] for l in range(2)])
    e2o, pepo = _hd(_pad_rows(edge_attr, e_pad), None,
                    jnp.stack([po['edge']['W']] * 2), None,
                    jnp.stack([po['edge']['b']] * 2),
                    _stack(oh, 'W_e'), 'leaky', 'tanh', a8oe)
    srcg_eo = idx2(edge_index[0], e_pad, n_pad)
    dst_eo = _pad1(edge_index[1], e_pad, n)
    ar_o = jnp.arange(n_pad, dtype=jnp.int32)
    srcg_no = jnp.stack([ar_o, ar_o + n_pad])
    dst_no = _pad1(batch, n_pad, nm)
    s_o, _ = _afp_core(oh, h0, hp0, e2o, _edge_pe(pepo, e, 2),
                       srcg_eo, dst_eo, srcg_no, dst_no, pool_pe(n_pad, n),
                       n_pad, g_pad, 256, 128, False)

    a8f0 = _a8([_stack(fh, 'atom', 0, 'a')[:, :_H, 0],
                _stack(fh, 'atom', 0, 'a')[:, _H:, 0]])
    h0f, hp0f = _hd(_pad_rows(frag_x, nf_pad), None, _stack(fh, 'W_in'), None,
                    jnp.zeros((2, _H), jnp.float32), None, 'tanh', 'none', a8f0)
    a8fe = _a8([_stack(fh, 'atom', l, 'a')[:, _H:, 0] for l in range(2)])
    e2f, pepf = _hd(_pad_rows(frag_edge_attr, ef_pad), None, _stack(fh, 'W_e'),
                    None, jnp.zeros((2, _H), jnp.float32), None, 'tanh', 'none', a8fe)
    srcg_ef = idx2(frag_edge_index[0], ef_pad, nf_pad)
    dst_ef = _pad1(frag_edge_index[1], ef_pad, nf)
    ar_f = jnp.arange(nf_pad, dtype=jnp.int32)
    srcg_nf = jnp.stack([ar_f, ar_f + nf_pad])
    dst_nf = _pad1(frag_batch, nf_pad, f)
    s_f, _ = _afp_core(fh, h0f, hp0f, e2f, _edge_pe(pepf, ef, 2),
                       srcg_ef, dst_ef, srcg_nf, dst_nf, pool_pe(nf_pad, nf),
                       nf_pad, f_pad, 128, 128, False)

    wtf = pf['att']['W']
    gf = _linN([s_f[0], s_f[1]],
               [fh[0]['W_out'] @ wtf[:_H], fh[1]['W_out'] @ wtf[_H:]],
               pf['att']['b'], 'relu')

    me = _linN([_pad_rows(motif_x, f_pad)], [pj['frag_lin']['W']],
               pj['frag_lin']['b'], 'leaky')
    a8j0 = _a8([_stack(jh, 'atom', 0, 'a')[:, :_H, 0],
                _stack(jh, 'atom', 0, 'a')[:, _H:, 0]])
    projw = jnp.stack([hp['proj']['W'] for hp in pj['heads']])
    projb = jnp.stack([hp['proj']['b'] for hp in pj['heads']])
    h0j, hp0j = _hd(gf, me, projw[:, :_H, :], projw[:, _H:, :], projb,
                    _stack(jh, 'W_in'), 'none', 'tanh', a8j0)
    a8je = _a8([_stack(jh, 'atom', l, 'a')[:, _H:, 0] for l in range(2)])
    e2j, pepj = _hd(_pad_rows(junction_edge_attr, ej_pad), None,
                    jnp.stack([pj['edge_lin']['W']] * 2), None,
                    jnp.stack([pj['edge_lin']['b']] * 2),
                    _stack(jh, 'W_e'), 'leaky', 'tanh', a8je)
    srcg_ej = idx2(junction_edge_index[0], ej_pad, f_pad)
    dst_ej = _pad1(junction_edge_index[1], ej_pad, f)
    ar_j = jnp.arange(f_pad, dtype=jnp.int32)
    srcg_nj = jnp.stack([ar_j, ar_j + f_pad])
    dst_nj = _pad1(junction_batch, f_pad, nm)
    s_j, alpha_j = _afp_core(jh, h0j, hp0j, e2j, _edge_pe(pepj, ej, 2),
                             srcg_ej, dst_ej, srcg_nj, dst_nj, pool_pe(f_pad, f),
                             f_pad, g_pad, 128, 128, True)

    sng = _linN([s_j[0], s_j[1]],
                [0.5 * jh[0]['W_out'], 0.5 * jh[1]['W_out']],
                jnp.zeros((_H,), jnp.float32), 'relu')

    wto = po['att']['W']
    go = _linN([s_o[0], s_o[1]],
               [oh[0]['W_out'] @ wto[:_H], oh[1]['W_out'] @ wto[_H:]],
               po['att']['b'], 'relu')

    gf2 = jnp.broadcast_to(gf[None], (2, f_pad, _H)).reshape(2 * f_pad, _H)
    acc_fr, _ = _seg_call(gf2, srcg_nj, dst_nj,
                          jnp.zeros((2, 16), jnp.float32),
                          jnp.zeros((2, 16), jnp.float32),
                          alpha_j, jnp.zeros((2, 16, _H), jnp.float32),
                          nseg_pad=g_pad, nsrc_off=f_pad, ch=128,
                          softmax=False, has_e=False, emit_alpha=False)

    w1, w2, w3 = pp['l1']['W'], pp['l2']['W'], pp['l3']['W']
    h1 = _linN([go, acc_fr[0], acc_fr[1], sng],
               [w1[:_H], 0.5 * w1[_H:2 * _H], 0.5 * w1[_H:2 * _H], w1[2 * _H:]],
               pp['l1']['b'], 'leaky3')
    h2 = _linN([h1], [w2], pp['l2']['b'], 'leaky3')
    w3p = jnp.zeros((w3.shape[0], _H), jnp.float32).at[:, :1].set(w3)
    b3p = jnp.zeros((_H,), jnp.float32).at[:1].set(pp['l3']['b'])
    out = _linN([h2], [w3p], b3p, 'none')
    return out[:nm, :1]


# speedup vs baseline: 3.9342x; 1.0361x over previous
import functools

import jax
import jax.numpy as jnp
from jax import lax
from jax.experimental import pallas as pl
from jax.experimental.pallas import tpu as pltpu
from jax.experimental.pallas import tpu_sc as plsc

_H = 128
_TM = 512
_NEG = -1e30


def _act(x, kind):
    if kind == 'none':
        return x
    if kind == 'leaky':
        return jnp.where(x >= 0, x, 0.01 * x)
    if kind == 'leaky3':
        return jnp.where(x >= 0, x, 0.001 * x)
    if kind == 'tanh':
        return jnp.tanh(x)
    if kind == 'relu':
        return jnp.maximum(x, 0.0)
    if kind == 'elu':
        return jnp.where(x > 0, x, jnp.exp(x) - 1.0)
    raise ValueError(kind)


def _pad_rows(a, m):
    return a if a.shape[0] == m else jnp.pad(a, ((0, m - a.shape[0]),) + ((0, 0),) * (a.ndim - 1))


def _pad1(a, m, val=0):
    return a if a.shape[0] == m else jnp.pad(a, (0, m - a.shape[0]), constant_values=val)


def _rup(n, m):
    return ((n + m - 1) // m) * m


def _linN_body(*refs, nx, act):
    xs = refs[:nx]
    ws = refs[nx:2 * nx]
    b = refs[2 * nx]
    y = refs[2 * nx + 1]
    acc = jnp.dot(xs[0][...], ws[0][...], preferred_element_type=jnp.float32, precision=lax.Precision.HIGHEST)
    for i in range(1, nx):
        acc = acc + jnp.dot(xs[i][...], ws[i][...], preferred_element_type=jnp.float32, precision=lax.Precision.HIGHEST)
    y[...] = _act(acc + b[...], act)


def _linN(xs, ws, b, act):
    m = xs[0].shape[0]
    hd = ws[0].shape[1]
    grid = (m // _TM,)
    in_specs = [pl.BlockSpec((_TM, x.shape[1]), lambda i: (i, 0)) for x in xs]
    in_specs += [pl.BlockSpec(w.shape, lambda i: (0, 0)) for w in ws]
    in_specs.append(pl.BlockSpec(b.shape, lambda i: (0,)))
    return pl.pallas_call(
        functools.partial(_linN_body, nx=len(xs), act=act),
        grid=grid,
        in_specs=in_specs,
        out_specs=pl.BlockSpec((_TM, hd), lambda i: (i, 0)),
        out_shape=jax.ShapeDtypeStruct((m, hd), jnp.float32),
    )(*xs, *ws, b)


def _hd_body(*refs, shared1, has2, shared2, act0, has_wc, act1):
    i = 0
    x1 = refs[i]; i += 1
    x2 = None
    if has2:
        x2 = refs[i]; i += 1
    w1 = refs[i]; i += 1
    w2 = None
    if has2:
        w2 = refs[i]; i += 1
    b = refs[i]; i += 1
    wc = None
    if has_wc:
        wc = refs[i]; i += 1
    a = refs[i]; i += 1
    y = refs[i]; p = refs[i + 1]
    x1v = x1[...] if shared1 else x1[0]
    acc = jnp.dot(x1v, w1[0], preferred_element_type=jnp.float32, precision=lax.Precision.HIGHEST)
    if has2:
        x2v = x2[...] if shared2 else x2[0]
        acc = acc + jnp.dot(x2v, w2[0], preferred_element_type=jnp.float32, precision=lax.Precision.HIGHEST)
    y0 = _act(acc + b[0, 0], act0)
    if has_wc:
        y0 = _act(jnp.dot(y0, wc[0], preferred_element_type=jnp.float32, precision=lax.Precision.HIGHEST), act1)
    y[...] = y0[None]
    p[...] = jnp.dot(y0, a[0], preferred_element_type=jnp.float32, precision=lax.Precision.HIGHEST)[None]


def _hd(x1, x2, w1, w2, b, wc, act0, act1, a):
    shared1 = x1.ndim == 2
    has2 = x2 is not None
    shared2 = has2 and x2.ndim == 2
    has_wc = wc is not None
    m = x1.shape[0] if shared1 else x1.shape[1]
    hout = wc.shape[2] if has_wc else w1.shape[2]
    grid = (2, m // _TM)

    in_specs = []
    args = []
    if shared1:
        k1 = x1.shape[1]
        in_specs.append(pl.BlockSpec((_TM, k1), lambda h, i: (i, 0)))
    else:
        k1 = x1.shape[2]
        in_specs.append(pl.BlockSpec((1, _TM, k1), lambda h, i: (h, i, 0)))
    args.append(x1)
    if has2:
        if shared2:
            k2 = x2.shape[1]
            in_specs.append(pl.BlockSpec((_TM, k2), lambda h, i: (i, 0)))
        else:
            k2 = x2.shape[2]
            in_specs.append(pl.BlockSpec((1, _TM, k2), lambda h, i: (h, i, 0)))
        args.append(x2)
    in_specs.append(pl.BlockSpec((1,) + w1.shape[1:], lambda h, i: (h, 0, 0)))
    args.append(w1)
    if has2:
        in_specs.append(pl.BlockSpec((1,) + w2.shape[1:], lambda h, i: (h, 0, 0)))
        args.append(w2)
    in_specs.append(pl.BlockSpec((1, 1, b.shape[1]), lambda h, i: (h, 0, 0)))
    args.append(b.reshape(2, 1, b.shape[1]))
    if has_wc:
        in_specs.append(pl.BlockSpec((1,) + wc.shape[1:], lambda h, i: (h, 0, 0)))
        args.append(wc)
    in_specs.append(pl.BlockSpec((1,) + a.shape[1:], lambda h, i: (h, 0, 0)))
    args.append(a)

    out_specs = [
        pl.BlockSpec((1, _TM, hout), lambda h, i: (h, i, 0)),
        pl.BlockSpec((1, _TM, 8), lambda h, i: (h, i, 0)),
    ]
    out_shape = [
        jax.ShapeDtypeStruct((2, m, hout), jnp.float32),
        jax.ShapeDtypeStruct((2, m, 8), jnp.float32),
    ]
    return pl.pallas_call(
        functools.partial(_hd_body, shared1=shared1, has2=has2, shared2=shared2,
                          act0=act0, has_wc=has_wc, act1=act1),
        grid=grid,
        in_specs=in_specs,
        out_specs=out_specs,
        out_shape=out_shape,
    )(*args)


def _seg_body(table_hbm, srcg_hbm, dst_hbm, pa_hbm, pb_hbm, pe_hbm, e_hbm,
              acc_out, alpha_out,
              pa_v, pb_v, ssum_v, w_v, srcg_c, dst_c, pe_c, alpha_c,
              rows_v, srcg_c2, dst_c2, alpha_c2, rows_v2,
              e_v, zbuf, acc_sh, ssum_sh, sem,
              *, ew, ch, nseg_pad, nsrc_off, softmax, has_e, emit_alpha):
    cid = lax.axis_index("c")
    sid = lax.axis_index("s")

    def zrow(j, c):
        def zcol(k, cc):
            zbuf[j, pl.ds(k * 16, 16)] = jnp.zeros((16,), jnp.float32)
            return cc
        return lax.fori_loop(0, 8, zcol, c)
    lax.fori_loop(0, 16, zrow, 0)

    rows_pt = nseg_pad // 16

    def zacc(j, c):
        pltpu.sync_copy(zbuf, acc_sh.at[pl.ds(sid * rows_pt + j * 16, 16)])
        return c
    lax.fori_loop(0, rows_pt // 16, zacc, 0)

    if softmax:
        def zs(j, c):
            pltpu.sync_copy(zbuf.at[0, pl.ds(0, 16)],
                            ssum_sh.at[pl.ds(sid * rows_pt + j * 16, 16)])
            return c
        lax.fori_loop(0, rows_pt // 16, zs, 0)
        pltpu.sync_copy(pa_hbm.at[cid, pl.ds(0, nseg_pad)], pa_v)
        pltpu.sync_copy(pb_hbm.at[cid, pl.ds(0, pb_hbm.shape[1])], pb_v)
    plsc.subcore_barrier()

    nch = ew // ch
    ebase0 = sid * ew
    off = cid * nsrc_off

    if softmax:
        def pass_a(c, carry):
            eb = ebase0 + c * ch
            pltpu.sync_copy(dst_hbm.at[pl.ds(eb, ch)], dst_c)
            pltpu.sync_copy(pe_hbm.at[cid, pl.ds(eb, ch)], pe_c)
            pltpu.sync_copy(srcg_hbm.at[cid, pl.ds(eb, ch)], srcg_c)

            def inner(j, cc):
                o = j * 16
                dv = dst_c[pl.ds(o, 16)]
                sv = srcg_c[pl.ds(o, 16)] - off
                l = (plsc.load_gather(pa_v, [dv])
                     + plsc.load_gather(pb_v, [sv]) + pe_c[pl.ds(o, 16)])
                l = jnp.where(l >= 0, l, l * 0.01)
                w_v[pl.ds(c * ch + o, 16)] = jnp.exp(l)
                return cc
            lax.fori_loop(0, ch // 16, inner, 0)
            pltpu.sync_copy(w_v.at[pl.ds(c * ch, ch)], ssum_sh.at[dst_c], add=True)
            return carry
        lax.fori_loop(0, nch, pass_a, 0)
        plsc.subcore_barrier()
        pltpu.sync_copy(ssum_sh, ssum_v)

    if softmax:
        def pass_b(c, carry):
            eb = ebase0 + c * ch
            pltpu.sync_copy(dst_hbm.at[pl.ds(eb, ch)], dst_c)
            pltpu.sync_copy(srcg_hbm.at[cid, pl.ds(eb, ch)], srcg_c)

            def mka(j, cc):
                o = j * 16
                w16 = w_v[pl.ds(c * ch + o, 16)]
                sv = plsc.load_gather(ssum_v, [dst_c[pl.ds(o, 16)]])
                alpha_c[pl.ds(o, 16)] = w16 / (sv + 1e-9)
                return cc
            lax.fori_loop(0, ch // 16, mka, 0)
            if emit_alpha:
                pltpu.sync_copy(alpha_c, alpha_out.at[cid, pl.ds(eb, ch)])

            pltpu.async_copy(table_hbm.at[srcg_c], rows_v, sem).wait()
            if has_e:
                pltpu.sync_copy(e_hbm.at[cid, pl.ds(eb, ch)], e_v)

            def srow(j, cc):
                av = plsc.load_gather(alpha_c, [jnp.full((16,), j, jnp.int32)])

                def scol(k, c2):
                    sl = pl.ds(k * 16, 16)
                    if has_e:
                        rows_v[j, sl] = (rows_v[j, sl] + e_v[j, sl]) * av
                    else:
                        rows_v[j, sl] = rows_v[j, sl] * av
                    return c2
                return lax.fori_loop(0, 8, scol, cc)
            lax.fori_loop(0, ch, srow, 0)
            pltpu.sync_copy(rows_v, acc_sh.at[dst_c], add=True)
            return carry
        lax.fori_loop(0, nch, pass_b, 0)
    else:
        def fetch_idx(c, sc, dc, ac):
            eb = ebase0 + c * ch
            pltpu.sync_copy(dst_hbm.at[pl.ds(eb, ch)], dc)
            pltpu.sync_copy(srcg_hbm.at[cid, pl.ds(eb, ch)], sc)
            pltpu.sync_copy(pe_hbm.at[cid, pl.ds(eb, ch)], ac)

        def process(c, sc, dc, ac, rv):
            pltpu.make_async_copy(table_hbm.at[sc], rv, sem).wait()
            eb = ebase0 + c * ch
            if has_e:
                pltpu.sync_copy(e_hbm.at[cid, pl.ds(eb, ch)], e_v)

            def srow(j, cc):
                av = plsc.load_gather(ac, [jnp.full((16,), j, jnp.int32)])

                def scol(k, c2):
                    sl = pl.ds(k * 16, 16)
                    if has_e:
                        rv[j, sl] = (rv[j, sl] + e_v[j, sl]) * av
                    else:
                        rv[j, sl] = rv[j, sl] * av
                    return c2
                return lax.fori_loop(0, 8, scol, cc)
            lax.fori_loop(0, ch, srow, 0)
            pltpu.sync_copy(rv, acc_sh.at[dc], add=True)

        if nch == 1:
            fetch_idx(0, srcg_c, dst_c, alpha_c)
            pltpu.async_copy(table_hbm.at[srcg_c], rows_v, sem)
            process(0, srcg_c, dst_c, alpha_c, rows_v)
        else:
            assert nch % 2 == 0
            fetch_idx(0, srcg_c, dst_c, alpha_c)
            pltpu.async_copy(table_hbm.at[srcg_c], rows_v, sem)

            def pair(c2, carry):
                c = 2 * c2
                fetch_idx(c + 1, srcg_c2, dst_c2, alpha_c2)
                pltpu.async_copy(table_hbm.at[srcg_c2], rows_v2, sem)
                process(c, srcg_c, dst_c, alpha_c, rows_v)

                @pl.when(c2 + 1 < nch // 2)
                def _():
                    fetch_idx(c + 2, srcg_c, dst_c, alpha_c)
                    pltpu.async_copy(table_hbm.at[srcg_c], rows_v, sem)
                process(c + 1, srcg_c2, dst_c2, alpha_c2, rows_v2)
                return carry
            lax.fori_loop(0, nch // 2, pair, 0)
    plsc.subcore_barrier()

    def dump(j, c):
        r = sid * rows_pt + j * 16
        pltpu.sync_copy(acc_sh.at[pl.ds(r, 16)], acc_out.at[cid, pl.ds(r, 16)])
        return c
    lax.fori_loop(0, rows_pt // 16, dump, 0)


def _seg_call(table2, srcg, dst, pa, pb, pe, e2, *, nseg_pad, nsrc_off, ch,
              softmax, has_e, emit_alpha):
    e_pad = dst.shape[0]
    ew = e_pad // 16
    d = _H
    out_type = [
        jax.ShapeDtypeStruct((2, nseg_pad, d), jnp.float32),
        jax.ShapeDtypeStruct((2, e_pad if emit_alpha else 16), jnp.float32),
    ]
    scratch = [
        pltpu.VMEM((nseg_pad if softmax else 16,), jnp.float32),
        pltpu.VMEM((pb.shape[1] if softmax else 16,), jnp.float32),
        pltpu.VMEM((nseg_pad if softmax else 16,), jnp.float32),
        pltpu.VMEM((ew if softmax else 16,), jnp.float32),
        pltpu.VMEM((ch,), jnp.int32),
        pltpu.VMEM((ch,), jnp.int32),
        pltpu.VMEM((ch if softmax else 16,), jnp.float32),
        pltpu.VMEM((ch,), jnp.float32),
        pltpu.VMEM((ch, d), jnp.float32),
        pltpu.VMEM((16 if softmax else ch,), jnp.int32),
        pltpu.VMEM((16 if softmax else ch,), jnp.int32),
        pltpu.VMEM((16 if softmax else ch,), jnp.float32),
        pltpu.VMEM((16, d) if softmax else (ch, d), jnp.float32),
        pltpu.VMEM((ch, d) if has_e else (16, d), jnp.float32),
        pltpu.VMEM((16, d), jnp.float32),
        pltpu.VMEM_SHARED((nseg_pad, d), jnp.float32),
        pltpu.VMEM_SHARED((nseg_pad if softmax else 16,), jnp.float32),
        pltpu.SemaphoreType.DMA,
    ]
    fn = pl.kernel(
        functools.partial(_seg_body, ew=ew, ch=ch, nseg_pad=nseg_pad,
                          nsrc_off=nsrc_off, softmax=softmax, has_e=has_e,
                          emit_alpha=emit_alpha),
        mesh=plsc.VectorSubcoreMesh(core_axis_name="c", subcore_axis_name="s"),
        out_type=out_type,
        scratch_types=scratch,
        compiler_params=pltpu.CompilerParams(needs_layout_passes=False),
    )
    return fn(table2, srcg, dst, pa, pb, pe, e2)


def _att_body(srcg_hbm, dst_hbm, pa_hbm, pb_hbm, pe_hbm,
              alpha_out,
              pa_v, pb_v, ssum_v, w_v, srcg_c, dst_c, pe_c, alpha_c, zbuf,
              ssum_sh,
              *, ew, ch, nseg_pad, nsrc_off):
    cid = lax.axis_index("c")
    sid = lax.axis_index("s")
    zbuf[pl.ds(0, 16)] = jnp.zeros((16,), jnp.float32)
    rows_pt = nseg_pad // 16

    def zs(j, c):
        pltpu.sync_copy(zbuf, ssum_sh.at[pl.ds(sid * rows_pt + j * 16, 16)])
        return c
    lax.fori_loop(0, rows_pt // 16, zs, 0)
    pltpu.sync_copy(pa_hbm.at[cid, pl.ds(0, nseg_pad)], pa_v)
    pltpu.sync_copy(pb_hbm.at[cid, pl.ds(0, pb_hbm.shape[1])], pb_v)
    plsc.subcore_barrier()

    nch = ew // ch
    ebase0 = sid * ew
    off = cid * nsrc_off

    def pass_a(c, carry):
        eb = ebase0 + c * ch
        pltpu.sync_copy(dst_hbm.at[pl.ds(eb, ch)], dst_c)
        pltpu.sync_copy(pe_hbm.at[cid, pl.ds(eb, ch)], pe_c)
        pltpu.sync_copy(srcg_hbm.at[cid, pl.ds(eb, ch)], srcg_c)

        def inner(j, cc):
            o = j * 16
            dv = dst_c[pl.ds(o, 16)]
            sv = srcg_c[pl.ds(o, 16)] - off
            l = (plsc.load_gather(pa_v, [dv])
                 + plsc.load_gather(pb_v, [sv]) + pe_c[pl.ds(o, 16)])
            l = jnp.where(l >= 0, l, l * 0.01)
            w_v[pl.ds(c * ch + o, 16)] = jnp.exp(l)
            return cc
        lax.fori_loop(0, ch // 16, inner, 0)
        pltpu.sync_copy(w_v.at[pl.ds(c * ch, ch)], ssum_sh.at[dst_c], add=True)
        return carry
    lax.fori_loop(0, nch, pass_a, 0)
    plsc.subcore_barrier()
    pltpu.sync_copy(ssum_sh, ssum_v)

    def norm(c, carry):
        eb = ebase0 + c * ch
        pltpu.sync_copy(dst_hbm.at[pl.ds(eb, ch)], dst_c)

        def mka(j, cc):
            o = j * 16
            w16 = w_v[pl.ds(c * ch + o, 16)]
            sv = plsc.load_gather(ssum_v, [dst_c[pl.ds(o, 16)]])
            alpha_c[pl.ds(o, 16)] = w16 / (sv + 1e-9)
            return cc
        lax.fori_loop(0, ch // 16, mka, 0)
        pltpu.sync_copy(alpha_c, alpha_out.at[cid, pl.ds(eb, ch)])
        return carry
    lax.fori_loop(0, nch, norm, 0)


def _att_call(srcg, dst, pa, pb, pe, *, nseg_pad, nsrc_off, ch):
    e_pad = dst.shape[0]
    ew = e_pad // 16
    scratch = [
        pltpu.VMEM((nseg_pad,), jnp.float32),
        pltpu.VMEM((pb.shape[1],), jnp.float32),
        pltpu.VMEM((nseg_pad,), jnp.float32),
        pltpu.VMEM((ew,), jnp.float32),
        pltpu.VMEM((ch,), jnp.int32),
        pltpu.VMEM((ch,), jnp.int32),
        pltpu.VMEM((ch,), jnp.float32),
        pltpu.VMEM((ch,), jnp.float32),
        pltpu.VMEM((16,), jnp.float32),
        pltpu.VMEM_SHARED((nseg_pad,), jnp.float32),
    ]
    fn = pl.kernel(
        functools.partial(_att_body, ew=ew, ch=ch, nseg_pad=nseg_pad,
                          nsrc_off=nsrc_off),
        mesh=plsc.VectorSubcoreMesh(core_axis_name="c", subcore_axis_name="s"),
        out_type=[jax.ShapeDtypeStruct((2, e_pad), jnp.float32)],
        scratch_types=scratch,
        compiler_params=pltpu.CompilerParams(needs_layout_passes=False),
    )
    res = fn(srcg, dst, pa, pb, pe)
    return res[0] if isinstance(res, (tuple, list)) else res


def _stack(heads, *path):
    def get(h):
        v = h
        for p in path:
            v = v[p]
        return v
    return jnp.stack([get(heads[0]), get(heads[1])])


def _a8(cols):
    a = jnp.zeros((2, _H, 8), jnp.float32)
    for i, c in enumerate(cols):
        a = a.at[:, :, i].set(c)
    return a


def _afp_core(heads, h, hp, e2, pe_list, srcg_e, dst_e, srcg_n, dst_n,
              pe_pool, n_pad, g_pad, ch_e, ch_n, emit_alpha):
    la = len(heads[0]['atom'])
    lm = len(heads[0]['mol'])
    dummy_e = jnp.zeros((2, 16, _H), jnp.float32)
    zerob = jnp.zeros((2, _H), jnp.float32)

    dummy16 = jnp.zeros((2, 16), jnp.float32)
    for l in range(la):
        if n_pad >= 8192:
            alpha_e = _att_call(srcg_e, dst_e, hp[:, :, 0], hp[:, :, 1],
                                pe_list[l], nseg_pad=n_pad, nsrc_off=n_pad,
                                ch=256)
            acc, _ = _seg_call(h.reshape(2 * n_pad, _H), srcg_e, dst_e,
                               dummy16, dummy16, alpha_e, e2,
                               nseg_pad=n_pad, nsrc_off=n_pad, ch=64,
                               softmax=False, has_e=True, emit_alpha=False)
        else:
            acc, _ = _seg_call(h.reshape(2 * n_pad, _H), srcg_e, dst_e,
                               hp[:, :, 0], hp[:, :, 1], pe_list[l], e2,
                               nseg_pad=n_pad, nsrc_off=n_pad, ch=ch_e,
                               softmax=True, has_e=True, emit_alpha=False)
        if l + 1 < la:
            nxt = _a8([_stack(heads, 'atom', l + 1, 'a')[:, :_H, 0],
                       _stack(heads, 'atom', l + 1, 'a')[:, _H:, 0]])
        else:
            nxt = _a8([_stack(heads, 'mol', ml, 'a')[:, _H:, 0] for ml in range(lm)])
        h, hp = _hd(h, acc, _stack(heads, 'atom', l, 'W_u'),
                    _stack(heads, 'atom', l, 'W_m'), zerob, None,
                    'elu', 'none', nxt)

    zs_g = jnp.zeros((2, g_pad), jnp.float32)
    zs_n = jnp.zeros((2, n_pad), jnp.float32)
    s, _ = _seg_call(h.reshape(2 * n_pad, _H), srcg_n, dst_n, zs_g, zs_n,
                     pe_pool, dummy_e, nseg_pad=g_pad, nsrc_off=n_pad,
                     ch=ch_n, softmax=True, has_e=False, emit_alpha=False)

    sp, _ = _hd(s, None, _a8([_stack(heads, 'mol', 0, 'a')[:, :_H, 0]]), None,
                jnp.zeros((2, 8), jnp.float32), None, 'none', 'none',
                jnp.zeros((2, 8, 8), jnp.float32))
    alpha = None
    for l in range(lm):
        acc_c, alpha = _seg_call(h.reshape(2 * n_pad, _H), srcg_n, dst_n,
                                 sp[:, :, 0], hp[:, :, l], pe_pool, dummy_e,
                                 nseg_pad=g_pad, nsrc_off=n_pad, ch=ch_n,
                                 softmax=True, has_e=False,
                                 emit_alpha=(emit_alpha and l == lm - 1))
        if l + 1 < lm:
            nxt = _a8([_stack(heads, 'mol', l + 1, 'a')[:, :_H, 0]])
        else:
            nxt = jnp.zeros((2, _H, 8), jnp.float32)
        s, sp = _hd(s, acc_c, _stack(heads, 'mol', l, 'W_s'),
                    _stack(heads, 'mol', l, 'W_c'), zerob, None,
                    'elu', 'none', nxt)
    return s, alpha


def _edge_pe(pep, e_real, la):
    ev = jnp.arange(pep.shape[1], dtype=jnp.int32) < e_real
    return [jnp.where(ev[None, :], pep[:, :, l], _NEG) for l in range(la)]


def kernel(x, edge_index, edge_attr, batch, frag_x, frag_edge_index,
           frag_edge_attr, frag_batch, motif_x, junction_edge_index,
           junction_edge_attr, junction_batch, params):
    n, e, nm = x.shape[0], edge_index.shape[1], 500
    nf, ef = frag_x.shape[0], frag_edge_index.shape[1]
    f, ej = motif_x.shape[0], junction_edge_index.shape[1]

    n_pad = _rup(n, 2048)
    e_pad = _rup(e, 16 * 256)
    nf_pad = _rup(nf, 2048)
    ef_pad = _rup(ef, 2048)
    f_pad = _rup(f, 2048)
    ej_pad = _rup(ej, 2048)
    g_pad = _rup(nm, 512)

    po, pf, pj, pp = params['origin'], params['frag'], params['junction'], params['pred']
    oh, fh = po['heads'], pf['heads']
    jh = [hp['afp'] for hp in pj['heads']]

    def idx2(src, m_pad, off):
        sp_ = _pad1(src, m_pad, 0)
        return jnp.stack([sp_, sp_ + off])

    def pool_pe(m_pad, real):
        v = jnp.arange(m_pad, dtype=jnp.int32) < real
        return jnp.broadcast_to(jnp.where(v, 0.0, _NEG)[None], (2, m_pad)).astype(jnp.float32)

    a8o0 = _a8([_stack(oh, 'atom', 0, 'a')[:, :_H, 0],
                _stack(oh, 'atom', 0, 'a')[:, _H:, 0]])
    h0, hp0 = _hd(_pad_rows(x, n_pad), None,
                  jnp.stack([po['node']['W']] * 2), None,
                  jnp.stack([po['node']['b']] * 2),
                  _stack(oh, 'W_in'), 'leaky', 'tanh', a8o0)
    a8oe = _a8([_stack(oh, 'atom', l, 'a')[:, _H:, 0] for l in range(2)])
    e2o, pepo = _hd(_pad_rows(edge_attr, e_pad), None,
                    jnp.stack([po['edge']['W']] * 2), None,
                    jnp.stack([po['edge']['b']] * 2),
                    _stack(oh, 'W_e'), 'leaky', 'tanh', a8oe)
    srcg_eo = idx2(edge_index[0], e_pad, n_pad)
    dst_eo = _pad1(edge_index[1], e_pad, n)
    ar_o = jnp.arange(n_pad, dtype=jnp.int32)
    srcg_no = jnp.stack([ar_o, ar_o + n_pad])
    dst_no = _pad1(batch, n_pad, nm)
    s_o, _ = _afp_core(oh, h0, hp0, e2o, _edge_pe(pepo, e, 2),
                       srcg_eo, dst_eo, srcg_no, dst_no, pool_pe(n_pad, n),
                       n_pad, g_pad, 256, 128, False)

    a8f0 = _a8([_stack(fh, 'atom', 0, 'a')[:, :_H, 0],
                _stack(fh, 'atom', 0, 'a')[:, _H:, 0]])
    h0f, hp0f = _hd(_pad_rows(frag_x, nf_pad), None, _stack(fh, 'W_in'), None,
                    jnp.zeros((2, _H), jnp.float32), None, 'tanh', 'none', a8f0)
    a8fe = _a8([_stack(fh, 'atom', l, 'a')[:, _H:, 0] for l in range(2)])
    e2f, pepf = _hd(_pad_rows(frag_edge_attr, ef_pad), None, _stack(fh, 'W_e'),
                    None, jnp.zeros((2, _H), jnp.float32), None, 'tanh', 'none', a8fe)
    srcg_ef = idx2(frag_edge_index[0], ef_pad, nf_pad)
    dst_ef = _pad1(frag_edge_index[1], ef_pad, nf)
    ar_f = jnp.arange(nf_pad, dtype=jnp.int32)
    srcg_nf = jnp.stack([ar_f, ar_f + nf_pad])
    dst_nf = _pad1(frag_batch, nf_pad, f)
    s_f, _ = _afp_core(fh, h0f, hp0f, e2f, _edge_pe(pepf, ef, 2),
                       srcg_ef, dst_ef, srcg_nf, dst_nf, pool_pe(nf_pad, nf),
                       nf_pad, f_pad, 128, 128, False)

    wtf = pf['att']['W']
    gf = _linN([s_f[0], s_f[1]],
               [fh[0]['W_out'] @ wtf[:_H], fh[1]['W_out'] @ wtf[_H:]],
               pf['att']['b'], 'relu')

    me = _linN([_pad_rows(motif_x, f_pad)], [pj['frag_lin']['W']],
               pj['frag_lin']['b'], 'leaky')
    a8j0 = _a8([_stack(jh, 'atom', 0, 'a')[:, :_H, 0],
                _stack(jh, 'atom', 0, 'a')[:, _H:, 0]])
    projw = jnp.stack([hp['proj']['W'] for hp in pj['heads']])
    projb = jnp.stack([hp['proj']['b'] for hp in pj['heads']])
    h0j, hp0j = _hd(gf, me, projw[:, :_H, :], projw[:, _H:, :], projb,
                    _stack(jh, 'W_in'), 'none', 'tanh', a8j0)
    a8je = _a8([_stack(jh, 'atom', l, 'a')[:, _H:, 0] for l in range(2)])
    e2j, pepj = _hd(_pad_rows(junction_edge_attr, ej_pad), None,
                    jnp.stack([pj['edge_lin']['W']] * 2), None,
                    jnp.stack([pj['edge_lin']['b']] * 2),
                    _stack(jh, 'W_e'), 'leaky', 'tanh', a8je)
    srcg_ej = idx2(junction_edge_index[0], ej_pad, f_pad)
    dst_ej = _pad1(junction_edge_index[1], ej_pad, f)
    ar_j = jnp.arange(f_pad, dtype=jnp.int32)
    srcg_nj = jnp.stack([ar_j, ar_j + f_pad])
    dst_nj = _pad1(junction_batch, f_pad, nm)
    s_j, alpha_j = _afp_core(jh, h0j, hp0j, e2j, _edge_pe(pepj, ej, 2),
                             srcg_ej, dst_ej, srcg_nj, dst_nj, pool_pe(f_pad, f),
                             f_pad, g_pad, 128, 128, True)

    sng = _linN([s_j[0], s_j[1]],
                [0.5 * jh[0]['W_out'], 0.5 * jh[1]['W_out']],
                jnp.zeros((_H,), jnp.float32), 'relu')

    wto = po['att']['W']
    go = _linN([s_o[0], s_o[1]],
               [oh[0]['W_out'] @ wto[:_H], oh[1]['W_out'] @ wto[_H:]],
               po['att']['b'], 'relu')

    gf2 = jnp.broadcast_to(gf[None], (2, f_pad, _H)).reshape(2 * f_pad, _H)
    acc_fr, _ = _seg_call(gf2, srcg_nj, dst_nj,
                          jnp.zeros((2, 16), jnp.float32),
                          jnp.zeros((2, 16), jnp.float32),
                          alpha_j, jnp.zeros((2, 16, _H), jnp.float32),
                          nseg_pad=g_pad, nsrc_off=f_pad, ch=128,
                          softmax=False, has_e=False, emit_alpha=False)

    w1, w2, w3 = pp['l1']['W'], pp['l2']['W'], pp['l3']['W']
    h1 = _linN([go, acc_fr[0], acc_fr[1], sng],
               [w1[:_H], 0.5 * w1[_H:2 * _H], 0.5 * w1[_H:2 * _H], w1[2 * _H:]],
               pp['l1']['b'], 'leaky3')
    h2 = _linN([h1], [w2], pp['l2']['b'], 'leaky3')
    w3p = jnp.zeros((w3.shape[0], _H), jnp.float32).at[:, :1].set(w3)
    b3p = jnp.zeros((_H,), jnp.float32).at[:1].set(pp['l3']['b'])
    out = _linN([h2], [w3p], b3p, 'none')
    return out[:nm, :1]
